# baseline (speedup 1.0000x reference)
import jax
import jax.numpy as jnp
from jax.experimental import pallas as pl
from jax.experimental.pallas import tpu as pltpu


def _pick_block(n: int) -> int:
    for r in range(min(n, 256), 7, -1):
        if n % r == 0 and r % 8 == 0:
            return r
    return n


def _body(adjT, adjB, x_ref, w1_ref, b1_ref, w2_ref, b2_ref, o_ref,
          xw_sc, hw_sc):
    p = pl.program_id(0)
    i = pl.program_id(1)
    r = adjT.shape[1]

    @pl.when(jnp.logical_and(p == 0, i == 0))
    def _init():
        xw_sc[...] = jnp.dot(x_ref[...], w1_ref[...],
                             preferred_element_type=jnp.float32)

    @pl.when(p == 0)
    def _layer1():
        for half, adj_ref in enumerate((adjT, adjB)):
            h = jnp.dot(adj_ref[0], xw_sc[...],
                        preferred_element_type=jnp.float32)
            h = jnp.maximum(h + b1_ref[...], 0.0)
            hw_sc[pl.ds((2 * i + half) * r, r), :] = jnp.dot(
                h, w2_ref[...], preferred_element_type=jnp.float32)

    @pl.when(p == 1)
    def _layer2():
        for half, adj_ref in enumerate((adjT, adjB)):
            logits = jnp.dot(adj_ref[0], hw_sc[...],
                             preferred_element_type=jnp.float32)
            logits = logits + b2_ref[...]
            m = jnp.max(logits, axis=-1, keepdims=True)
            s = logits - m
            lse = jnp.log(jnp.sum(jnp.exp(s), axis=-1, keepdims=True))
            o_ref[pl.ds(half * r, r), :] = s - lse


def kernel(x, adjs, W1, b1, W2, b2):
    n, in_feats = x.shape
    h_feats = W1.shape[1]
    num_classes = W2.shape[1]
    r = _pick_block(n)
    nb = n // (2 * r)
    b1r = b1.reshape(1, h_feats)
    b2r = b2.reshape(1, num_classes)

    return pl.pallas_call(
        _body,
        grid=(2, nb),
        in_specs=[
            pl.BlockSpec((1, r, n), lambda p, i: (p, 2 * i, 0)),
            pl.BlockSpec((1, r, n), lambda p, i: (p, 2 * i + 1, 0)),
            pl.BlockSpec((n, in_feats), lambda p, i: (0, 0)),
            pl.BlockSpec((in_feats, h_feats), lambda p, i: (0, 0)),
            pl.BlockSpec((1, h_feats), lambda p, i: (0, 0)),
            pl.BlockSpec((h_feats, num_classes), lambda p, i: (0, 0)),
            pl.BlockSpec((1, num_classes), lambda p, i: (0, 0)),
        ],
        out_specs=pl.BlockSpec((2 * r, num_classes), lambda p, i: (p * i, 0)),
        out_shape=jax.ShapeDtypeStruct((n, num_classes), jnp.float32),
        scratch_shapes=[
            pltpu.VMEM((n, h_feats), jnp.float32),
            pltpu.VMEM((n, num_classes), jnp.float32),
        ],
        compiler_params=pltpu.CompilerParams(
            dimension_semantics=("arbitrary", "arbitrary"),
            vmem_limit_bytes=100 * 1024 * 1024,
        ),
    )(adjs, adjs, x, W1, b1r, W2, b2r)


# speedup vs baseline: 1.0457x; 1.0457x over previous
import jax
import jax.numpy as jnp
from jax.experimental import pallas as pl
from jax.experimental.pallas import tpu as pltpu


def _pick_block(n: int) -> int:
    for r in range(min(n, 512), 7, -1):
        if n % r == 0 and r % 8 == 0:
            return r
    return n


def _make_body(nb):
    def _body(adj_ref, x_ref, w1_ref, b1_ref, w2_ref, b2_ref, o_ref,
              xw_sc, hw_sc):
        c = pl.program_id(0)
        p = c // nb
        i = c - p * nb
        r = adj_ref.shape[1]

        @pl.when(c == 0)
        def _init():
            xw_sc[...] = jnp.dot(x_ref[...], w1_ref[...],
                                 preferred_element_type=jnp.float32)

        @pl.when(p == 0)
        def _layer1():
            h = jnp.dot(adj_ref[0], xw_sc[...],
                        preferred_element_type=jnp.float32)
            h = jnp.maximum(h + b1_ref[...], 0.0)
            hw_sc[pl.ds(i * r, r), :] = jnp.dot(
                h, w2_ref[...], preferred_element_type=jnp.float32)

        @pl.when(p == 1)
        def _layer2():
            logits = jnp.dot(adj_ref[0], hw_sc[...],
                             preferred_element_type=jnp.float32)
            logits = logits + b2_ref[...]
            m = jnp.max(logits, axis=-1, keepdims=True)
            s = logits - m
            lse = jnp.log(jnp.sum(jnp.exp(s), axis=-1, keepdims=True))
            o_ref[...] = s - lse

    return _body


def kernel(x, adjs, W1, b1, W2, b2):
    n, in_feats = x.shape
    h_feats = W1.shape[1]
    num_classes = W2.shape[1]
    r = _pick_block(n)
    nb = n // r
    b1r = b1.reshape(1, h_feats)
    b2r = b2.reshape(1, num_classes)

    return pl.pallas_call(
        _make_body(nb),
        grid=(2 * nb,),
        in_specs=[
            pl.BlockSpec((1, r, n), lambda c: (c // nb, c % nb, 0)),
            pl.BlockSpec((n, in_feats), lambda c: (0, 0)),
            pl.BlockSpec((in_feats, h_feats), lambda c: (0, 0)),
            pl.BlockSpec((1, h_feats), lambda c: (0, 0)),
            pl.BlockSpec((h_feats, num_classes), lambda c: (0, 0)),
            pl.BlockSpec((1, num_classes), lambda c: (0, 0)),
        ],
        out_specs=pl.BlockSpec((r, num_classes),
                               lambda c: ((c // nb) * (c % nb), 0)),
        out_shape=jax.ShapeDtypeStruct((n, num_classes), jnp.float32),
        scratch_shapes=[
            pltpu.VMEM((n, h_feats), jnp.float32),
            pltpu.VMEM((n, num_classes), jnp.float32),
        ],
        compiler_params=pltpu.CompilerParams(
            dimension_semantics=("arbitrary",),
            vmem_limit_bytes=100 * 1024 * 1024,
        ),
    )(adjs, x, W1, b1r, W2, b2r)


# speedup vs baseline: 1.0494x; 1.0036x over previous
import jax
import jax.numpy as jnp
from jax.experimental import pallas as pl
from jax.experimental.pallas import tpu as pltpu


def _pick_block(n: int) -> int:
    for r in range(min(n, 512), 7, -1):
        if n % r == 0 and r % 8 == 0:
            return r
    return n


def _body(adj_ref, x_ref, w1_ref, b1_ref, w2_ref, b2_ref, o_ref,
          xw_sc, hw_sc):
    p = pl.program_id(0)
    i = pl.program_id(1)
    r = adj_ref.shape[1]

    @pl.when(jnp.logical_and(p == 0, i == 0))
    def _init():
        xw_sc[...] = jnp.dot(x_ref[...], w1_ref[...],
                             preferred_element_type=jnp.float32)

    @pl.when(p == 0)
    def _layer1():
        h = jnp.dot(adj_ref[0], xw_sc[...], preferred_element_type=jnp.float32)
        h = jnp.maximum(h + b1_ref[...], 0.0)
        hw_sc[pl.ds(i * r, r), :] = jnp.dot(h, w2_ref[...],
                                            preferred_element_type=jnp.float32)

    @pl.when(p == 1)
    def _layer2():
        logits = jnp.dot(adj_ref[0], hw_sc[...],
                         preferred_element_type=jnp.float32)
        logits = logits + b2_ref[...]
        m = jnp.max(logits, axis=-1, keepdims=True)
        s = logits - m
        lse = jnp.log(jnp.sum(jnp.exp(s), axis=-1, keepdims=True))
        o_ref[...] = s - lse


def kernel(x, adjs, W1, b1, W2, b2):
    n, in_feats = x.shape
    h_feats = W1.shape[1]
    num_classes = W2.shape[1]
    r = _pick_block(n)
    b1r = b1.reshape(1, h_feats)
    b2r = b2.reshape(1, num_classes)

    return pl.pallas_call(
        _body,
        grid=(2, n // r),
        in_specs=[
            pl.BlockSpec((1, r, n), lambda p, i: (p, i, 0)),
            pl.BlockSpec((n, in_feats), lambda p, i: (0, 0)),
            pl.BlockSpec((in_feats, h_feats), lambda p, i: (0, 0)),
            pl.BlockSpec((1, h_feats), lambda p, i: (0, 0)),
            pl.BlockSpec((h_feats, num_classes), lambda p, i: (0, 0)),
            pl.BlockSpec((1, num_classes), lambda p, i: (0, 0)),
        ],
        out_specs=pl.BlockSpec((r, num_classes), lambda p, i: (p * i, 0)),
        out_shape=jax.ShapeDtypeStruct((n, num_classes), jnp.float32),
        scratch_shapes=[
            pltpu.VMEM((n, h_feats), jnp.float32),
            pltpu.VMEM((n, num_classes), jnp.float32),
        ],
        compiler_params=pltpu.CompilerParams(
            dimension_semantics=("arbitrary", "arbitrary"),
            vmem_limit_bytes=100 * 1024 * 1024,
        ),
    )(adjs, x, W1, b1r, W2, b2r)
